# baseline (device time: 29716 ns/iter reference)
import jax
import jax.numpy as jnp
from jax import lax
from jax.experimental import pallas as pl
from jax.experimental.pallas import tpu as pltpu


def kernel(x):
    _, m, n = x.shape
    half = n // 2

    half_m = m // 2

    def body(x_ref, out_ref, recv_buf, send_sems, recv_sems):
        my_x = lax.axis_index("x")
        my_y = lax.axis_index("y")
        my_z = lax.axis_index("z")
        other_x = 1 - my_x

        barrier_sem = pltpu.get_barrier_semaphore()
        pl.semaphore_signal(
            barrier_sem,
            inc=1,
            device_id=(other_x, my_y, my_z),
            device_id_type=pl.DeviceIdType.MESH,
        )
        pl.semaphore_wait(barrier_sem, 1)

        rdmas = []
        for s in range(2):
            rdma = pltpu.make_async_remote_copy(
                src_ref=x_ref.at[0, pl.ds(s * half_m, half_m),
                                 pl.ds(other_x * half, half)],
                dst_ref=recv_buf.at[pl.ds(s * half_m, half_m), :],
                send_sem=send_sems.at[s],
                recv_sem=recv_sems.at[s],
                device_id=(other_x, my_y, my_z),
                device_id_type=pl.DeviceIdType.MESH,
            )
            rdma.start()
            rdmas.append(rdma)
        for rdma in rdmas:
            rdma.wait()

        out_ref[:, :] = x_ref[0, :, pl.ds(my_x * half, half)] + recv_buf[:, :]

    return pl.pallas_call(
        body,
        out_shape=jax.ShapeDtypeStruct((m, half), x.dtype),
        in_specs=[pl.BlockSpec(memory_space=pltpu.VMEM)],
        out_specs=pl.BlockSpec(memory_space=pltpu.VMEM),
        scratch_shapes=[
            pltpu.VMEM((m, half), x.dtype),
            pltpu.SemaphoreType.DMA((2,)),
            pltpu.SemaphoreType.DMA((2,)),
        ],
        compiler_params=pltpu.CompilerParams(collective_id=0),
    )(x)


# device time: 22352 ns/iter; 1.3295x vs baseline; 1.3295x over previous
import jax
import jax.numpy as jnp
from jax import lax
from jax.experimental import pallas as pl
from jax.experimental.pallas import tpu as pltpu

K = 8


def kernel(x):
    _, m, n = x.shape
    half = n // 2
    half_m = m // 2
    chunk_m = half_m // K

    def body(x_ref, out_ref, xrecv, yrecv, xsend_sems, xrecv_sems,
             ysend_sems, yrecv_sems):
        my_x = lax.axis_index("x")
        my_y = lax.axis_index("y")
        my_z = lax.axis_index("z")
        other_x = 1 - my_x
        other_y = 1 - my_y

        barrier_sem = pltpu.get_barrier_semaphore()
        for nbr in [(other_x, my_y, my_z), (my_x, other_y, my_z)]:
            pl.semaphore_signal(
                barrier_sem, inc=1,
                device_id=nbr, device_id_type=pl.DeviceIdType.MESH,
            )
        pl.semaphore_wait(barrier_sem, 2)

        x_rdmas = []
        for k in range(K):
            rdma = pltpu.make_async_remote_copy(
                src_ref=x_ref.at[0,
                                 pl.ds(my_y * half_m + k * chunk_m, chunk_m),
                                 pl.ds(other_x * half, half)],
                dst_ref=xrecv.at[k],
                send_sem=xsend_sems.at[k],
                recv_sem=xrecv_sems.at[k],
                device_id=(other_x, my_y, my_z),
                device_id_type=pl.DeviceIdType.MESH,
            )
            rdma.start()
            x_rdmas.append(rdma)

        y_rdmas = []
        for k in range(K):
            x_rdmas[k].wait_recv()
            fwd = pltpu.make_async_remote_copy(
                src_ref=xrecv.at[k],
                dst_ref=yrecv.at[k],
                send_sem=ysend_sems.at[k],
                recv_sem=yrecv_sems.at[k],
                device_id=(my_x, other_y, my_z),
                device_id_type=pl.DeviceIdType.MESH,
            )
            fwd.start()
            y_rdmas.append(fwd)
            rows = pl.ds(my_y * half_m + k * chunk_m, chunk_m)
            out_ref[rows, :] = (
                x_ref[0, rows, pl.ds(my_x * half, half)] + xrecv[k]
            )

        for k in range(K):
            y_rdmas[k].wait_recv()
            rows = pl.ds(other_y * half_m + k * chunk_m, chunk_m)
            out_ref[rows, :] = (
                x_ref[0, rows, pl.ds(my_x * half, half)] + yrecv[k]
            )

        for k in range(K):
            x_rdmas[k].wait_send()
            y_rdmas[k].wait_send()

    return pl.pallas_call(
        body,
        out_shape=jax.ShapeDtypeStruct((m, half), x.dtype),
        in_specs=[pl.BlockSpec(memory_space=pltpu.VMEM)],
        out_specs=pl.BlockSpec(memory_space=pltpu.VMEM),
        scratch_shapes=[
            pltpu.VMEM((K, chunk_m, half), x.dtype),
            pltpu.VMEM((K, chunk_m, half), x.dtype),
            pltpu.SemaphoreType.DMA((K,)),
            pltpu.SemaphoreType.DMA((K,)),
            pltpu.SemaphoreType.DMA((K,)),
            pltpu.SemaphoreType.DMA((K,)),
        ],
        compiler_params=pltpu.CompilerParams(collective_id=0),
    )(x)
